# baseline (speedup 1.0000x reference)
import functools

import jax
import jax.numpy as jnp
from jax import lax
from jax.experimental import pallas as pl
from jax.experimental.pallas import tpu as pltpu
from jax.experimental.pallas import tpu_sc as plsc

_L = 1_000_000
_NW = 32
_WPW = 124_928
_CW = 2_048
_LAST_W = _L - 7 * _WPW


def _sc_body(src, dst, buf0, buf1, tbuf, s0, s1, o0, o1):
    wid = lax.axis_index("s") * 2 + lax.axis_index("c")
    blk = wid // 8
    k = wid % 8
    base = k * _WPW

    bufs = (buf0, buf1)
    isems = (s0, s1)
    osems = (o0, o1)

    n_full = _WPW // _CW
    rows = pl.ds(blk * 8, 8)

    def in_copy(j, w, slot):
        return pltpu.make_async_copy(
            src.at[rows, pl.ds(base + j * _CW, w)],
            bufs[slot].at[:, pl.ds(0, w)],
            isems[slot],
        )

    def out_copy(j, w, slot):
        return pltpu.make_async_copy(
            bufs[slot].at[:, pl.ds(0, w)],
            dst.at[rows, pl.ds(base + j * _CW, w)],
            osems[slot],
        )

    in_copy(0, _CW, 0).start()
    for j in range(n_full):
        slot = j % 2
        nslot = 1 - slot
        if j + 1 < n_full:
            if j + 1 >= 2:
                out_copy(j - 1, _CW, nslot).wait()
            in_copy(j + 1, _CW, nslot).start()
        in_copy(j, _CW, slot).wait()
        out_copy(j, _CW, slot).start()
    out_copy(n_full - 2, _CW, (n_full - 2) % 2).wait()
    out_copy(n_full - 1, _CW, (n_full - 1) % 2).wait()

    tail = _LAST_W - n_full * _CW

    del tail


def kernel(embeds):
    t = embeds.T
    mesh = plsc.VectorSubcoreMesh(core_axis_name="c", subcore_axis_name="s")
    f = functools.partial(
        pl.kernel,
        mesh=mesh,
        out_type=jax.ShapeDtypeStruct(t.shape, t.dtype),
        scratch_types=[
            pltpu.VMEM((8, _CW), jnp.float32),
            pltpu.VMEM((8, _CW), jnp.float32),
            pltpu.VMEM((8, _LAST_W - (_WPW // _CW) * _CW), jnp.float32),
            pltpu.SemaphoreType.DMA,
            pltpu.SemaphoreType.DMA,
            pltpu.SemaphoreType.DMA,
            pltpu.SemaphoreType.DMA,
        ],
    )(_sc_body)
    return f(t).T


# speedup vs baseline: 1.3990x; 1.3990x over previous
import jax
import jax.numpy as jnp
from jax.experimental import pallas as pl
from jax.experimental.pallas import tpu as pltpu

_L = 1_000_000
_BIG = 65_536
_SMALL = 8_192
_K = 24
_DI = 12


def _block_widths(i):
    if i == 0:
        return [_SMALL] * 8 + [_BIG] * 14 + [_L - 8 * _SMALL - 14 * _BIG]
    if i == 3:
        return [_BIG] * 14 + [_SMALL] * 8 + [_L - 8 * _SMALL - 14 * _BIG]
    return [_BIG] * 15 + [_L - 15 * _BIG]


_CHUNKS = []
_ODD = []
for _i in range(4):
    _off = 0
    for _w in _block_widths(_i):
        (_CHUNKS if _w % 128 == 0 else _ODD).append((_i, _off, _w))
        _off += _w
    assert _off == _L
assert len(_ODD) == 4 and all(_w == _ODD[0][2] for (_, _, _w) in _ODD)
_ODD_W = _ODD[0][2]


def _copy_body(src, dst, bufs, tbufs, in_sems, out_sems, tin_sems, tout_sems):
    s3 = src.reshape(4, 8, _L)
    d3 = dst.reshape(4, 8, _L)
    n_chunks = len(_CHUNKS)

    def in_copy(c):
        i, off, w = _CHUNKS[c]
        return pltpu.make_async_copy(
            s3.at[i, :, pl.ds(off, w)],
            bufs.at[c % _K, :, pl.ds(0, w)],
            in_sems.at[c % _K],
        )

    def out_copy(c):
        i, off, w = _CHUNKS[c]
        return pltpu.make_async_copy(
            bufs.at[c % _K, :, pl.ds(0, w)],
            d3.at[i, :, pl.ds(off, w)],
            out_sems.at[c % _K],
        )

    def tin_copy(j):
        i, off, w = _ODD[j]
        return pltpu.make_async_copy(
            s3.at[i, :, pl.ds(off, w)], tbufs.at[j], tin_sems.at[j]
        )

    def tout_copy(j):
        i, off, w = _ODD[j]
        return pltpu.make_async_copy(
            tbufs.at[j], d3.at[i, :, pl.ds(off, w)], tout_sems.at[j]
        )

    for j in range(4):
        tin_copy(j).start()
    for c in range(min(_DI, n_chunks)):
        in_copy(c).start()
    waited = set()
    for c in range(n_chunks):
        in_copy(c).wait()
        out_copy(c).start()
        p = c + _DI
        if p < n_chunks:
            if p >= _K:
                out_copy(p - _K).wait()
                waited.add(p - _K)
            in_copy(p).start()
    for j in range(4):
        tin_copy(j).wait()
        tout_copy(j).start()
    for c in range(n_chunks):
        if c not in waited:
            out_copy(c).wait()
    for j in range(4):
        tout_copy(j).wait()


def kernel(embeds):
    t = embeds.T
    out = pl.pallas_call(
        _copy_body,
        out_shape=jax.ShapeDtypeStruct(t.shape, t.dtype),
        in_specs=[pl.BlockSpec(memory_space=pltpu.MemorySpace.HBM)],
        out_specs=pl.BlockSpec(memory_space=pltpu.MemorySpace.HBM),
        scratch_shapes=[
            pltpu.VMEM((_K, 8, _BIG), jnp.float32),
            pltpu.VMEM((4, 8, _ODD_W), jnp.float32),
            pltpu.SemaphoreType.DMA((_K,)),
            pltpu.SemaphoreType.DMA((_K,)),
            pltpu.SemaphoreType.DMA((4,)),
            pltpu.SemaphoreType.DMA((4,)),
        ],
    )(t)
    return out.T


# speedup vs baseline: 1.4035x; 1.0032x over previous
import jax
import jax.numpy as jnp
from jax.experimental import pallas as pl
from jax.experimental.pallas import tpu as pltpu

_L = 1_000_000
_BIG = 65_536
_SMALL = 8_192
_TINY = 2_048
_K = 24
_DI = 12


def _block_widths(i):
    odd = _L - 4 * _TINY - 7 * _SMALL - 14 * _BIG
    if i == 0:
        return [_TINY] * 4 + [_SMALL] * 7 + [_BIG] * 14 + [odd]
    if i == 3:
        return [_BIG] * 14 + [_SMALL] * 7 + [_TINY] * 4 + [odd]
    return [_BIG] * 15 + [_L - 15 * _BIG]


_CHUNKS = []
_ODD = []
for _i in range(4):
    _off = 0
    for _w in _block_widths(_i):
        (_CHUNKS if _w % 128 == 0 else _ODD).append((_i, _off, _w))
        _off += _w
    assert _off == _L
assert len(_ODD) == 4 and all(_w == _ODD[0][2] for (_, _, _w) in _ODD)
_ODD_W = _ODD[0][2]


def _copy_body(src, dst, bufs, tbufs, in_sems, out_sems, tin_sems, tout_sems):
    s3 = src.reshape(4, 8, _L)
    d3 = dst.reshape(4, 8, _L)
    n_chunks = len(_CHUNKS)

    def in_copy(c):
        i, off, w = _CHUNKS[c]
        return pltpu.make_async_copy(
            s3.at[i, :, pl.ds(off, w)],
            bufs.at[c % _K, :, pl.ds(0, w)],
            in_sems.at[c % _K],
        )

    def out_copy(c):
        i, off, w = _CHUNKS[c]
        return pltpu.make_async_copy(
            bufs.at[c % _K, :, pl.ds(0, w)],
            d3.at[i, :, pl.ds(off, w)],
            out_sems.at[c % _K],
        )

    def tin_copy(j):
        i, off, w = _ODD[j]
        return pltpu.make_async_copy(
            s3.at[i, :, pl.ds(off, w)], tbufs.at[j], tin_sems.at[j]
        )

    def tout_copy(j):
        i, off, w = _ODD[j]
        return pltpu.make_async_copy(
            tbufs.at[j], d3.at[i, :, pl.ds(off, w)], tout_sems.at[j]
        )

    for c in range(min(_DI, n_chunks)):
        in_copy(c).start()
    for j in range(4):
        tin_copy(j).start()
    waited = set()
    for c in range(n_chunks):
        in_copy(c).wait()
        out_copy(c).start()
        p = c + _DI
        if p < n_chunks:
            if p >= _K:
                out_copy(p - _K).wait()
                waited.add(p - _K)
            in_copy(p).start()
        if c == n_chunks // 2:
            for j in range(4):
                tin_copy(j).wait()
                tout_copy(j).start()
    for c in range(n_chunks):
        if c not in waited:
            out_copy(c).wait()
    for j in range(4):
        tout_copy(j).wait()


def kernel(embeds):
    t = embeds.T
    out = pl.pallas_call(
        _copy_body,
        out_shape=jax.ShapeDtypeStruct(t.shape, t.dtype),
        in_specs=[pl.BlockSpec(memory_space=pltpu.MemorySpace.HBM)],
        out_specs=pl.BlockSpec(memory_space=pltpu.MemorySpace.HBM),
        scratch_shapes=[
            pltpu.VMEM((_K, 8, _BIG), jnp.float32),
            pltpu.VMEM((4, 8, _ODD_W), jnp.float32),
            pltpu.SemaphoreType.DMA((_K,)),
            pltpu.SemaphoreType.DMA((_K,)),
            pltpu.SemaphoreType.DMA((4,)),
            pltpu.SemaphoreType.DMA((4,)),
        ],
    )(t)
    return out.T
